# baseline (speedup 1.0000x reference)
import functools

import jax
import jax.numpy as jnp
from jax.experimental import pallas as pl
from jax.experimental.pallas import tpu as pltpu

N = 4096
K = 4
D = 256
H = 256
BN = 512
NB = N // BN

_SELU_ALPHA = 1.6732632423543772
_SELU_SCALE = 1.0507009873554805


def _selu(v):
    return _SELU_SCALE * jnp.where(v > 0, v, _SELU_ALPHA * (jnp.exp(v) - 1.0))


def _mm_t(a, w):
    return jax.lax.dot_general(a, w, (((1,), (1,)), ((), ())),
                               preferred_element_type=jnp.float32)


def _layer_body(adj_ref, x_ref, wih_ref, whh_ref, bih_ref, bhh_ref,
                g_ref, b_ref, out_ref, hx_ref):
    k = pl.program_id(1)
    x = x_ref[...]
    a = adj_ref[0]
    hx_ref[pl.ds(k * BN, BN), :] = _selu(
        jnp.dot(a, x, preferred_element_type=jnp.float32))

    @pl.when(k == K - 1)
    def _tail():
        hx = hx_ref[...]
        gi = _mm_t(hx, wih_ref[...]) + bih_ref[...]
        bhh = bhh_ref[...]
        h = jnp.zeros((BN, H), dtype=jnp.float32)
        s = jnp.zeros((BN, H), dtype=jnp.float32)
        for t in range(K):
            git = gi[t * BN:(t + 1) * BN]
            if t == 0:
                gh = jnp.broadcast_to(bhh, (BN, 3 * H))
            else:
                gh = _mm_t(h, whh_ref[...]) + bhh
            r = jax.nn.sigmoid(git[:, 0:H] + gh[:, 0:H])
            z = jax.nn.sigmoid(git[:, H:2 * H] + gh[:, H:2 * H])
            n = jnp.tanh(git[:, 2 * H:] + r * gh[:, 2 * H:])
            h = (1.0 - z) * n + z * h
            s = s + h
        mu = jnp.mean(s, axis=-1, keepdims=True)
        var = jnp.mean((s - mu) ** 2, axis=-1, keepdims=True)
        out_ref[...] = (s - mu) * jax.lax.rsqrt(var + 1e-5) * g_ref[...] + b_ref[...]


@functools.partial(jax.jit, static_argnames=())
def _diffusion_layer(x, adj_list, wih, whh, bih, bhh, g, b):
    return pl.pallas_call(
        _layer_body,
        grid=(NB, K),
        in_specs=[
            pl.BlockSpec((1, BN, N), lambda i, k: (k, i, 0)),
            pl.BlockSpec((N, D), lambda i, k: (0, 0)),
            pl.BlockSpec((3 * H, D), lambda i, k: (0, 0)),
            pl.BlockSpec((3 * H, H), lambda i, k: (0, 0)),
            pl.BlockSpec((1, 3 * H), lambda i, k: (0, 0)),
            pl.BlockSpec((1, 3 * H), lambda i, k: (0, 0)),
            pl.BlockSpec((1, H), lambda i, k: (0, 0)),
            pl.BlockSpec((1, H), lambda i, k: (0, 0)),
        ],
        out_specs=pl.BlockSpec((BN, H), lambda i, k: (i, 0)),
        out_shape=jax.ShapeDtypeStruct((N, H), jnp.float32),
        scratch_shapes=[pltpu.VMEM((K * BN, D), jnp.float32)],
    )(adj_list, x, wih, whh, bih, bhh, g, b)


def kernel(x, adj_list, W_ih0, W_hh0, b_ih0, b_hh0, ln_g0, ln_b0,
           W_ih1, W_hh1, b_ih1, b_hh1, ln_g1, ln_b1):
    h = _diffusion_layer(x, adj_list, W_ih0, W_hh0,
                         b_ih0.reshape(1, -1), b_hh0.reshape(1, -1),
                         ln_g0.reshape(1, -1), ln_b0.reshape(1, -1))
    h = _diffusion_layer(h, adj_list, W_ih1, W_hh1,
                         b_ih1.reshape(1, -1), b_hh1.reshape(1, -1),
                         ln_g1.reshape(1, -1), ln_b1.reshape(1, -1))
    return h


# speedup vs baseline: 1.1132x; 1.1132x over previous
import functools

import jax
import jax.numpy as jnp
from jax.experimental import pallas as pl
from jax.experimental.pallas import tpu as pltpu

N = 4096
K = 4
D = 256
H = 256
BN = 256
NB = N // BN

_SELU_ALPHA = 1.6732632423543772
_SELU_SCALE = 1.0507009873554805


def _selu(v):
    return _SELU_SCALE * jnp.where(v > 0, v, _SELU_ALPHA * (jnp.exp(v) - 1.0))


def _mm_t(a, w):
    return jax.lax.dot_general(a, w, (((1,), (1,)), ((), ())),
                               preferred_element_type=jnp.float32)


def _layer_body(adj_ref, x_ref, wih_ref, whh_ref, bih_ref, bhh_ref,
                g_ref, b_ref, out_ref):
    x = x_ref[...]
    a = adj_ref[...].reshape(K * BN, N)
    hx = _selu(jnp.dot(a, x, preferred_element_type=jnp.float32))
    gi = _mm_t(hx, wih_ref[...]) + bih_ref[...]

    bhh = bhh_ref[...]
    h = jnp.zeros((BN, H), dtype=jnp.float32)
    s = jnp.zeros((BN, H), dtype=jnp.float32)
    for t in range(K):
        git = gi[t * BN:(t + 1) * BN]
        if t == 0:
            gh = jnp.broadcast_to(bhh, (BN, 3 * H))
        else:
            gh = _mm_t(h, whh_ref[...]) + bhh
        r = jax.nn.sigmoid(git[:, 0:H] + gh[:, 0:H])
        z = jax.nn.sigmoid(git[:, H:2 * H] + gh[:, H:2 * H])
        n = jnp.tanh(git[:, 2 * H:] + r * gh[:, 2 * H:])
        h = (1.0 - z) * n + z * h
        s = s + h

    mu = jnp.mean(s, axis=-1, keepdims=True)
    var = jnp.mean((s - mu) ** 2, axis=-1, keepdims=True)
    out_ref[...] = (s - mu) * jax.lax.rsqrt(var + 1e-5) * g_ref[...] + b_ref[...]


@functools.partial(jax.jit, static_argnames=())
def _diffusion_layer(x, adj_list, wih, whh, bih, bhh, g, b):
    return pl.pallas_call(
        _layer_body,
        grid=(NB,),
        in_specs=[
            pl.BlockSpec((K, BN, N), lambda i: (0, i, 0)),
            pl.BlockSpec((N, D), lambda i: (0, 0)),
            pl.BlockSpec((3 * H, D), lambda i: (0, 0)),
            pl.BlockSpec((3 * H, H), lambda i: (0, 0)),
            pl.BlockSpec((1, 3 * H), lambda i: (0, 0)),
            pl.BlockSpec((1, 3 * H), lambda i: (0, 0)),
            pl.BlockSpec((1, H), lambda i: (0, 0)),
            pl.BlockSpec((1, H), lambda i: (0, 0)),
        ],
        out_specs=pl.BlockSpec((BN, H), lambda i: (i, 0)),
        out_shape=jax.ShapeDtypeStruct((N, H), jnp.float32),
        compiler_params=pltpu.CompilerParams(
            dimension_semantics=("parallel",),
        ),
    )(adj_list, x, wih, whh, bih, bhh, g, b)


def kernel(x, adj_list, W_ih0, W_hh0, b_ih0, b_hh0, ln_g0, ln_b0,
           W_ih1, W_hh1, b_ih1, b_hh1, ln_g1, ln_b1):
    h = _diffusion_layer(x, adj_list, W_ih0, W_hh0,
                         b_ih0.reshape(1, -1), b_hh0.reshape(1, -1),
                         ln_g0.reshape(1, -1), ln_b0.reshape(1, -1))
    h = _diffusion_layer(h, adj_list, W_ih1, W_hh1,
                         b_ih1.reshape(1, -1), b_hh1.reshape(1, -1),
                         ln_g1.reshape(1, -1), ln_b1.reshape(1, -1))
    return h
